# Initial kernel scaffold; baseline (speedup 1.0000x reference)
import jax
import jax.numpy as jnp
from jax.experimental import pallas as pl


def kernel(x, table):
    raise NotImplementedError("write your pallas kernel here")


# speedup vs baseline: 1.4823x; 1.4823x over previous
import functools

import jax
import jax.numpy as jnp
from jax import lax
from jax.experimental import pallas as pl
from jax.experimental.pallas import tpu as pltpu
from jax.experimental.pallas import tpu_sc as plsc


def _make_gather(B, D, n_workers, chunk):
    b_per_w = B // n_workers
    n_chunks = b_per_w // chunk
    mesh = plsc.VectorSubcoreMesh(core_axis_name="c", subcore_axis_name="s")
    nc = plsc.get_sparse_core_info().num_cores

    @functools.partial(
        pl.kernel,
        mesh=mesh,
        out_type=jax.ShapeDtypeStruct((B, D), jnp.float32),
        scratch_types=[
            pltpu.VMEM((b_per_w,), jnp.int32),
            pltpu.VMEM((chunk, D), jnp.float32),
            pltpu.SemaphoreType.DMA,
        ],
        compiler_params=pltpu.CompilerParams(use_tc_tiling_on_sc=False),
    )
    def gather_kernel(idx_hbm, table_hbm, out_hbm, idx_v, rows_v, sem):
        wid = lax.axis_index("s") * nc + lax.axis_index("c")
        base = wid * b_per_w
        pltpu.sync_copy(idx_hbm.at[pl.ds(base, b_per_w)], idx_v)

        def body(i, carry):
            off = i * chunk
            pltpu.async_copy(
                table_hbm.at[idx_v.at[pl.ds(off, chunk)]], rows_v, sem
            ).wait()
            pltpu.sync_copy(rows_v, out_hbm.at[pl.ds(base + off, chunk)])
            return carry

        lax.fori_loop(0, n_chunks, body, 0)

    return gather_kernel


def kernel(x, table):
    B = x.shape[0] * x.shape[1]
    D = table.shape[1]
    flat_idx = x.reshape(B)
    out = _make_gather(B, D, n_workers=32, chunk=1280)(flat_idx, table)
    return out.reshape(x.shape[0], x.shape[1], D)


# speedup vs baseline: 1.5019x; 1.0133x over previous
import functools

import jax
import jax.numpy as jnp
from jax import lax
from jax.experimental import pallas as pl
from jax.experimental.pallas import tpu as pltpu
from jax.experimental.pallas import tpu_sc as plsc


def _make_gather(B, D, n_workers, chunk, nbuf=2):
    b_per_w = B // n_workers
    n_chunks = b_per_w // chunk
    assert n_chunks % nbuf == 0 and n_chunks >= 2 * nbuf
    mesh = plsc.VectorSubcoreMesh(core_axis_name="c", subcore_axis_name="s")
    nc = plsc.get_sparse_core_info().num_cores

    @functools.partial(
        pl.kernel,
        mesh=mesh,
        out_type=jax.ShapeDtypeStruct((B, D), jnp.float32),
        scratch_types=[
            pltpu.VMEM((b_per_w,), jnp.int32),
            [pltpu.VMEM((chunk, D), jnp.float32) for _ in range(nbuf)],
            [pltpu.SemaphoreType.DMA for _ in range(nbuf)],
            [pltpu.SemaphoreType.DMA for _ in range(nbuf)],
        ],
        compiler_params=pltpu.CompilerParams(use_tc_tiling_on_sc=False),
    )
    def gather_kernel(idx_hbm, table_hbm, out_hbm, idx_v, rows, gsem, wsem):
        wid = lax.axis_index("s") * nc + lax.axis_index("c")
        base = wid * b_per_w
        pltpu.sync_copy(idx_hbm.at[pl.ds(base, b_per_w)], idx_v)

        def start_gather(i, b):
            pltpu.async_copy(
                table_hbm.at[idx_v.at[pl.ds(i * chunk, chunk)]], rows[b], gsem[b]
            )

        for b in range(nbuf):
            start_gather(b, b)

        def body(g, carry):
            for b in range(nbuf):
                i = g * nbuf + b
                pltpu.make_async_copy(
                    table_hbm.at[pl.ds(0, chunk)], rows[b], gsem[b]
                ).wait()
                copy = pltpu.async_copy(
                    rows[b], out_hbm.at[pl.ds(base + i * chunk, chunk)], wsem[b]
                )
                copy.wait()
                @pl.when(g < n_chunks // nbuf - 1)
                def _():
                    start_gather(i + nbuf, b)

            return carry

        lax.fori_loop(0, n_chunks // nbuf, body, 0)

    return gather_kernel


def kernel(x, table):
    B = x.shape[0] * x.shape[1]
    D = table.shape[1]
    flat_idx = x.reshape(B)
    out = _make_gather(B, D, n_workers=32, chunk=1280)(flat_idx, table)
    return out.reshape(x.shape[0], x.shape[1], D)
